# baseline (speedup 1.0000x reference)
import functools

import jax
import jax.numpy as jnp
from jax import lax
from jax.experimental import pallas as pl
from jax.experimental.pallas import tpu as pltpu
from jax.experimental.pallas import tpu_sc as plsc

B = 16384
D = 32
NC = 2
NS = 16
NW = NC * NS
BPW = B // NW
ICHUNK = 128
NCHUNK = BPW // ICHUNK


def _body(uid_hbm, mid_hbm, ut_hbm, mt_hbm, wb_hbm, out_hbm,
          uidx_v, midx_v, urows_v, mrows_v, wb_v, psum_v, out_v, usem, msem):
    wid = lax.axis_index("s") * NC + lax.axis_index("c")
    base = wid * BPW

    pltpu.sync_copy(uid_hbm.at[pl.ds(wid * NCHUNK, NCHUNK), :], uidx_v)
    pltpu.sync_copy(mid_hbm.at[pl.ds(wid * NCHUNK, NCHUNK), :], midx_v)
    pltpu.sync_copy(wb_hbm, wb_v)

    for j in range(NCHUNK):
        pltpu.async_copy(ut_hbm.at[uidx_v.at[j]],
                         urows_v.at[pl.ds(j * ICHUNK, ICHUNK)], usem)
        pltpu.async_copy(mt_hbm.at[midx_v.at[j]],
                         mrows_v.at[pl.ds(j * ICHUNK, ICHUNK)], msem)
    for j in range(NCHUNK):
        pltpu.make_async_copy(ut_hbm.at[uidx_v.at[j]],
                              urows_v.at[pl.ds(j * ICHUNK, ICHUNK)], usem).wait()
        pltpu.make_async_copy(mt_hbm.at[midx_v.at[j]],
                              mrows_v.at[pl.ds(j * ICHUNK, ICHUNK)], msem).wait()

    w0 = wb_v[pl.ds(0, 16)]
    w1 = wb_v[pl.ds(16, 16)]
    bias = wb_v[pl.ds(32, 16)]

    def step(b, _):
        u0 = urows_v[b, pl.ds(0, 16)]
        u1 = urows_v[b, pl.ds(16, 16)]
        m0 = mrows_v[b, pl.ds(0, 16)]
        m1 = mrows_v[b, pl.ds(16, 16)]
        t = u0 * m0 * w0 + u1 * m1 * w1
        psum_v[b, :] = plsc.cumsum(t)
        return 0

    lax.fori_loop(0, BPW, step, 0, unroll=8)

    lanes = lax.iota(jnp.int32, 16)
    last = jnp.full((16,), 15, jnp.int32)

    def collect(c, _):
        g = plsc.load_gather(psum_v, [c * 16 + lanes, last])
        out_v[pl.ds(c * 16, 16)] = g + bias
        return 0

    lax.fori_loop(0, BPW // 16, collect, 0, unroll=4)

    pltpu.sync_copy(out_v, out_hbm.at[pl.ds(base, BPW)])


@jax.jit
def _run(user_id, movie_id, user_table, movie_table, wb):
    mesh = plsc.VectorSubcoreMesh(core_axis_name="c", subcore_axis_name="s",
                                  num_cores=NC, num_subcores=NS)
    f = pl.kernel(
        _body,
        out_type=jax.ShapeDtypeStruct((B,), jnp.float32),
        mesh=mesh,
        compiler_params=pltpu.CompilerParams(needs_layout_passes=False,
                                             use_tc_tiling_on_sc=False),
        scratch_types=[
            pltpu.VMEM((NCHUNK, ICHUNK), jnp.int32),
            pltpu.VMEM((NCHUNK, ICHUNK), jnp.int32),
            pltpu.VMEM((BPW, D), jnp.float32),
            pltpu.VMEM((BPW, D), jnp.float32),
            pltpu.VMEM((48,), jnp.float32),
            pltpu.VMEM((BPW, 16), jnp.float32),
            pltpu.VMEM((BPW,), jnp.float32),
            pltpu.SemaphoreType.DMA,
            pltpu.SemaphoreType.DMA,
        ],
    )
    return f(user_id, movie_id, user_table, movie_table, wb)


def kernel(user_id, movie_id, user_table, movie_table, dense_w, dense_b):
    wb = jnp.concatenate(
        [dense_w.reshape(D), jnp.broadcast_to(dense_b, (16,))])
    out = _run(user_id.reshape(NW * NCHUNK, ICHUNK),
               movie_id.reshape(NW * NCHUNK, ICHUNK),
               user_table, movie_table, wb)
    return out.reshape(B, 1)


# speedup vs baseline: 3.1303x; 3.1303x over previous
import functools

import jax
import jax.numpy as jnp
from jax import lax
from jax.experimental import pallas as pl
from jax.experimental.pallas import tpu as pltpu
from jax.experimental.pallas import tpu_sc as plsc

B = 16384
D = 32
NC = 2
NS = 16
NW = NC * NS
BPW = B // NW
ICHUNK = 128
NCHUNK = BPW // ICHUNK
NBUF = 8

_MESH = plsc.VectorSubcoreMesh(core_axis_name="c", subcore_axis_name="s",
                               num_cores=NC, num_subcores=NS)


def _stage_a(uid_hbm, ut_hbm, w_hbm, urows_hbm,
             uidx_v, tiles_v, w_v, urows_v, *sems):
    wid = lax.axis_index("s") * NC + lax.axis_index("c")

    pltpu.sync_copy(uid_hbm.at[pl.ds(wid * NCHUNK, NCHUNK), :], uidx_v)
    pltpu.sync_copy(w_hbm, w_v)

    w0 = w_v[pl.ds(0, 16)]
    w1 = w_v[pl.ds(16, 16)]

    d_lo = lax.iota(jnp.int32, 16)
    rb_lo = d_lo // 8
    sub_lo = d_lo % 8
    rb_hi = rb_lo + 2

    NGRP = BPW // 16

    def load_ids(g):
        return uidx_v[g // 8, pl.ds((g % 8) * 16, 16)]

    def fire(uid, slot):
        cb = pl.multiple_of((uid >> 7) * 128, 128)
        for rb in range(4):
            pltpu.async_copy(ut_hbm.at[pl.ds(rb * 8, 8), pl.ds(cb, 128)],
                             tiles_v.at[slot, rb], sems[slot])

    def extract(uid, i, slot):
        lane = jnp.full((16,), uid & 127, jnp.int32)
        slot_v = jnp.full((16,), slot, jnp.int32)
        cb = pl.multiple_of((uid >> 7) * 128, 128)
        for rb in range(4):
            pltpu.make_async_copy(ut_hbm.at[pl.ds(rb * 8, 8), pl.ds(cb, 128)],
                                  tiles_v.at[slot, rb], sems[slot]).wait()
        u0 = plsc.load_gather(tiles_v, [slot_v, rb_lo, sub_lo, lane])
        u1 = plsc.load_gather(tiles_v, [slot_v, rb_lo + 2, sub_lo, lane])
        urows_v[i, pl.ds(0, 16)] = u0 * w0
        urows_v[i, pl.ds(16, 16)] = u1 * w1

    ids0 = load_ids(0)
    for k in range(NBUF):
        fire(ids0[k], k)

    def step(g, _):
        ids_g = load_ids(g)
        for k in range(NBUF):
            extract(ids_g[k], g * 16 + k, k)
            fire(ids_g[k + 8], k)
        for k in range(NBUF):
            extract(ids_g[k + 8], g * 16 + k + 8, k)

            @pl.when(g + 1 < NGRP)
            def _():
                ids_n = load_ids(jnp.minimum(g + 1, NGRP - 1))
                fire(ids_n[k], k)
        return 0

    lax.fori_loop(0, NGRP, step, 0)

    pltpu.sync_copy(urows_v, urows_hbm.at[pl.ds(wid * BPW, BPW), :])


def _stage_b(mid_hbm, mt_hbm, urows_hbm, wb_hbm, out_hbm,
             midx_v, mrows_v, urows_v, wb_v, psum_v, out_v, msem, usem):
    wid = lax.axis_index("s") * NC + lax.axis_index("c")
    base = wid * BPW

    pltpu.sync_copy(mid_hbm.at[pl.ds(wid * NCHUNK, NCHUNK), :], midx_v)
    pltpu.sync_copy(wb_hbm, wb_v)

    for j in range(NCHUNK):
        pltpu.async_copy(mt_hbm.at[midx_v.at[j]],
                         mrows_v.at[pl.ds(j * ICHUNK, ICHUNK)], msem)
    cp_u = pltpu.async_copy(urows_hbm.at[pl.ds(base, BPW), :], urows_v, usem)
    for j in range(NCHUNK):
        pltpu.make_async_copy(mt_hbm.at[midx_v.at[j]],
                              mrows_v.at[pl.ds(j * ICHUNK, ICHUNK)], msem).wait()
    cp_u.wait()

    bias = wb_v[pl.ds(32, 16)]

    def step(b, _):
        u0 = urows_v[b, pl.ds(0, 16)]
        u1 = urows_v[b, pl.ds(16, 16)]
        m0 = mrows_v[b, pl.ds(0, 16)]
        m1 = mrows_v[b, pl.ds(16, 16)]
        t = u0 * m0 + u1 * m1
        psum_v[b, :] = plsc.cumsum(t)
        return 0

    lax.fori_loop(0, BPW, step, 0, unroll=8)

    lanes = lax.iota(jnp.int32, 16)
    last = jnp.full((16,), 15, jnp.int32)

    def collect(c, _):
        g = plsc.load_gather(psum_v, [c * 16 + lanes, last])
        out_v[pl.ds(c * 16, 16)] = g + bias
        return 0

    lax.fori_loop(0, BPW // 16, collect, 0, unroll=4)

    pltpu.sync_copy(out_v, out_hbm.at[pl.ds(base, BPW)])


@jax.jit
def _run(user_id, movie_id, user_table_t, movie_table, wb):
    fa = pl.kernel(
        _stage_a,
        out_type=jax.ShapeDtypeStruct((B, D), jnp.float32),
        mesh=_MESH,
        compiler_params=pltpu.CompilerParams(needs_layout_passes=False,
                                             use_tc_tiling_on_sc=True),
        scratch_types=[
            pltpu.VMEM((NCHUNK, ICHUNK), jnp.int32),
            pltpu.VMEM((NBUF, 4, 8, 128), jnp.float32),
            pltpu.VMEM((48,), jnp.float32),
            pltpu.VMEM((BPW, D), jnp.float32),
        ] + [pltpu.SemaphoreType.DMA] * NBUF,
    )
    urows = fa(user_id, user_table_t, wb)

    fb = pl.kernel(
        _stage_b,
        out_type=jax.ShapeDtypeStruct((B,), jnp.float32),
        mesh=_MESH,
        compiler_params=pltpu.CompilerParams(needs_layout_passes=False,
                                             use_tc_tiling_on_sc=False),
        scratch_types=[
            pltpu.VMEM((NCHUNK, ICHUNK), jnp.int32),
            pltpu.VMEM((BPW, D), jnp.float32),
            pltpu.VMEM((BPW, D), jnp.float32),
            pltpu.VMEM((48,), jnp.float32),
            pltpu.VMEM((BPW, 16), jnp.float32),
            pltpu.VMEM((BPW,), jnp.float32),
            pltpu.SemaphoreType.DMA,
            pltpu.SemaphoreType.DMA,
        ],
    )
    return fb(movie_id, movie_table, urows, wb)


def kernel(user_id, movie_id, user_table, movie_table, dense_w, dense_b):
    wb = jnp.concatenate(
        [dense_w.reshape(D), jnp.broadcast_to(dense_b, (16,))])
    out = _run(user_id.reshape(NW * NCHUNK, ICHUNK),
               movie_id.reshape(NW * NCHUNK, ICHUNK),
               user_table.T, movie_table, wb)
    return out.reshape(B, 1)
